# baseline (speedup 1.0000x reference)
import functools

import jax
import jax.numpy as jnp
from jax import lax
from jax.experimental import pallas as pl
from jax.experimental.pallas import tpu as pltpu
from jax.experimental.pallas import tpu_sc as plsc

MIN_FLOAT = -3.4028234663852886e+36

_NW = 32


def _make_sc_gather_a(V, D, RA):
    ra = RA // _NW
    mesh = plsc.VectorSubcoreMesh(core_axis_name="c", subcore_axis_name="s")

    @functools.partial(
        pl.kernel,
        mesh=mesh,
        out_type=[jax.ShapeDtypeStruct((RA, D), jnp.float32)],
        scratch_types=[
            pltpu.VMEM((ra,), jnp.int32),
            pltpu.VMEM((ra, D), jnp.float32),
            pltpu.SemaphoreType.DMA,
        ],
    )
    def sc_a(ids_hbm, table_hbm, out_hbm, idx_v, rows_v, sem):
        wid = lax.axis_index("s") * 2 + lax.axis_index("c")
        base = wid * ra
        pltpu.sync_copy(ids_hbm.at[pl.ds(base, ra)], idx_v)
        pltpu.async_copy(table_hbm.at[idx_v], rows_v, sem).wait()
        pltpu.sync_copy(rows_v, out_hbm.at[pl.ds(base, ra)])

    return sc_a


def _make_sc_gather_b(V, D, B, RB):
    bp = B // _NW
    rb = RB // _NW
    nd = D // 16
    mesh = plsc.VectorSubcoreMesh(core_axis_name="c", subcore_axis_name="s")

    @functools.partial(
        pl.kernel,
        mesh=mesh,
        out_type=[
            jax.ShapeDtypeStruct((B,), jnp.float32),
            jax.ShapeDtypeStruct((RB, D), jnp.float32),
        ],
        scratch_types=[
            pltpu.VMEM((bp,), jnp.int32),
            pltpu.VMEM((rb,), jnp.int32),
            pltpu.VMEM((bp, D), jnp.float32),
            pltpu.VMEM((bp, D), jnp.float32),
            pltpu.VMEM((rb, D), jnp.float32),
            pltpu.VMEM((bp,), jnp.float32),
            pltpu.SemaphoreType.DMA,
            pltpu.SemaphoreType.DMA,
            pltpu.SemaphoreType.DMA,
            pltpu.SemaphoreType.DMA,
        ],
    )
    def sc_b(tgt_hbm, ids_hbm, x_hbm, table_hbm, pos_hbm, negw_hbm,
             tidx_v, nidx_v, xin_v, prow_v, nrow_v, pos_v,
             sem, sem_i, sem_x, sem_st):
        wid = lax.axis_index("s") * 2 + lax.axis_index("c")
        pbase = wid * bp
        nbase = wid * rb
        cx = pltpu.async_copy(x_hbm.at[pl.ds(pbase, bp)], xin_v, sem_x)
        ci1 = pltpu.async_copy(tgt_hbm.at[pl.ds(pbase, bp)], tidx_v, sem_i)
        ci2 = pltpu.async_copy(ids_hbm.at[pl.ds(nbase, rb)], nidx_v, sem_i)
        ci1.wait()
        ci2.wait()
        c1 = pltpu.async_copy(table_hbm.at[tidx_v], prow_v, sem)
        c2 = pltpu.async_copy(table_hbm.at[nidx_v], nrow_v, sem)
        c1.wait()
        c2.wait()
        cst = pltpu.async_copy(nrow_v, negw_hbm.at[pl.ds(nbase, rb)], sem_st)
        cx.wait()

        lanes = lax.iota(jnp.int32, 16)
        gdn = lax.GatherDimensionNumbers(
            offset_dims=(), collapsed_slice_dims=(0,), start_index_map=(0,))
        perms = [(lanes ^ sh)[:, None] for sh in (8, 4, 2, 1)]

        def group_dot(g, _):
            vec = jnp.zeros((16,), jnp.float32)
            for j in range(16):
                r = g * 16 + j
                acc = prow_v[r, pl.ds(0, 16)] * xin_v[r, pl.ds(0, 16)]
                for c in range(1, nd):
                    acc = acc + (prow_v[r, pl.ds(c * 16, 16)]
                                 * xin_v[r, pl.ds(c * 16, 16)])
                for p in perms:
                    acc = acc + lax.gather(
                        acc, p, dimension_numbers=gdn, slice_sizes=(1,),
                        mode=lax.GatherScatterMode.PROMISE_IN_BOUNDS)
                vec = jnp.where(lanes == j, acc, vec)
            pos_v[pl.ds(g * 16, 16)] = vec
            return _

        lax.fori_loop(0, bp // 16, group_dot, 0)
        pltpu.sync_copy(pos_v, pos_hbm.at[pl.ds(pbase, bp)])
        cst.wait()

    return sc_b


def _tc_a_body(mids_ref, tgt_ref, x_ref, nw_ref, out_ref):
    scores = lax.dot_general(nw_ref[...], x_ref[...], (((1,), (1,)), ((), ())),
                             preferred_element_type=jnp.float32)
    mask = mids_ref[...] == tgt_ref[...]
    out_ref[...] = jnp.where(mask, MIN_FLOAT, scores)


def _tc_b_body(mids_ref, tgt_ref, pos_ref, x_ref, nw_ref, prev_ref, out_ref,
               *, bn, nb):
    j = pl.program_id(0)
    scores = lax.dot_general(nw_ref[...], x_ref[...], (((1,), (1,)), ((), ())),
                             preferred_element_type=jnp.float32)
    mask = mids_ref[...] == tgt_ref[...]
    scores = jnp.where(mask, MIN_FLOAT, scores)
    row0 = jnp.where(j == nb - 1, 0, (j + 8) * bn)
    grow = row0 + lax.broadcasted_iota(jnp.int32, scores.shape, 0)
    scores = jnp.where(grow == 0, pos_ref[...], scores)
    out_ref[...] = scores


def kernel(inputs, targets, neg_ids, table):
    B, D = inputs.shape
    V = table.shape[0]
    N = neg_ids.shape[0]
    BN = 256
    tgt = targets.astype(jnp.int32)
    nid = neg_ids.astype(jnp.int32)

    NPAD = ((N + 1 + 255) // 256) * 256
    tail = NPAD - 1 - N
    nid_pad = jnp.concatenate(
        [jnp.zeros((1,), jnp.int32), nid, jnp.zeros((tail,), jnp.int32)])
    mids_col = jnp.concatenate(
        [jnp.full((1,), -1, jnp.int32), nid, jnp.full((tail,), -1, jnp.int32)]
    ).reshape(NPAD, 1)

    RA = 7 * BN
    ids_a = lax.slice(nid_pad, (BN,), (BN + RA,))
    ids_b = jnp.concatenate(
        [lax.slice(nid_pad, (BN + RA,), (NPAD,)), lax.slice(nid_pad, (0,), (BN,))])
    RB = NPAD - RA

    res_a = _make_sc_gather_a(V, D, RA)(ids_a, table)
    negw_a = res_a[0] if isinstance(res_a, (list, tuple)) else res_a
    pos, negw_b = _make_sc_gather_b(V, D, B, RB)(tgt, ids_b, inputs, table)

    tgt_row = tgt.reshape(1, B)
    pos_row = pos.reshape(1, B)
    n_out = N + 1

    out_a = pl.pallas_call(
        _tc_a_body,
        grid=(RA // BN,),
        in_specs=[
            pl.BlockSpec((BN, 1), lambda j: (j + 1, 0)),
            pl.BlockSpec(memory_space=pltpu.VMEM),
            pl.BlockSpec(memory_space=pltpu.VMEM),
            pl.BlockSpec((BN, D), lambda j: (j, 0)),
        ],
        out_specs=pl.BlockSpec((BN, B), lambda j: (j + 1, 0)),
        out_shape=jax.ShapeDtypeStruct((n_out, B), jnp.float32),
        compiler_params=pltpu.CompilerParams(
            dimension_semantics=("arbitrary",)),
    )(mids_col, tgt_row, inputs, negw_a)

    nb = RB // BN
    out_t = pl.pallas_call(
        functools.partial(_tc_b_body, bn=BN, nb=nb),
        grid=(nb,),
        in_specs=[
            pl.BlockSpec((BN, 1),
                         lambda j: (jnp.where(j == nb - 1, 0, j + 8), 0)),
            pl.BlockSpec(memory_space=pltpu.VMEM),
            pl.BlockSpec(memory_space=pltpu.VMEM),
            pl.BlockSpec(memory_space=pltpu.VMEM),
            pl.BlockSpec((BN, D), lambda j: (j, 0)),
            pl.BlockSpec(memory_space=pl.ANY),
        ],
        out_specs=pl.BlockSpec((BN, B),
                               lambda j: (jnp.where(j == nb - 1, 0, j + 8), 0)),
        out_shape=jax.ShapeDtypeStruct((n_out, B), jnp.float32),
        input_output_aliases={5: 0},
        compiler_params=pltpu.CompilerParams(
            dimension_semantics=("arbitrary",)),
    )(mids_col, tgt_row, pos_row, inputs, negw_b, out_a)

    return out_t.T


# speedup vs baseline: 1.0172x; 1.0172x over previous
import functools

import jax
import jax.numpy as jnp
from jax import lax
from jax.experimental import pallas as pl
from jax.experimental.pallas import tpu as pltpu
from jax.experimental.pallas import tpu_sc as plsc

MIN_FLOAT = -3.4028234663852886e+36

_NW = 32


def _make_sc_gather_a(V, D, RA):
    ra = RA // _NW
    mesh = plsc.VectorSubcoreMesh(core_axis_name="c", subcore_axis_name="s")

    @functools.partial(
        pl.kernel,
        mesh=mesh,
        out_type=[jax.ShapeDtypeStruct((RA, D), jnp.float32)],
        scratch_types=[
            pltpu.VMEM((ra,), jnp.int32),
            pltpu.VMEM((ra, D), jnp.float32),
            pltpu.SemaphoreType.DMA,
        ],
    )
    def sc_a(ids_hbm, table_hbm, out_hbm, idx_v, rows_v, sem):
        wid = lax.axis_index("s") * 2 + lax.axis_index("c")
        base = wid * ra
        pltpu.sync_copy(ids_hbm.at[pl.ds(base, ra)], idx_v)
        pltpu.async_copy(table_hbm.at[idx_v], rows_v, sem).wait()
        pltpu.sync_copy(rows_v, out_hbm.at[pl.ds(base, ra)])

    return sc_a


def _make_sc_gather_b(V, D, B, RB):
    bp = B // _NW
    rb = RB // _NW
    nd = D // 16
    mesh = plsc.VectorSubcoreMesh(core_axis_name="c", subcore_axis_name="s")

    @functools.partial(
        pl.kernel,
        mesh=mesh,
        out_type=[
            jax.ShapeDtypeStruct((B,), jnp.float32),
            jax.ShapeDtypeStruct((RB, D), jnp.float32),
        ],
        scratch_types=[
            pltpu.VMEM((bp,), jnp.int32),
            pltpu.VMEM((rb,), jnp.int32),
            pltpu.VMEM((bp, D), jnp.float32),
            pltpu.VMEM((bp, D), jnp.float32),
            pltpu.VMEM((rb, D), jnp.float32),
            pltpu.VMEM((bp,), jnp.float32),
            pltpu.SemaphoreType.DMA,
            pltpu.SemaphoreType.DMA,
            pltpu.SemaphoreType.DMA,
            pltpu.SemaphoreType.DMA,
        ],
    )
    def sc_b(tgt_hbm, ids_hbm, x_hbm, table_hbm, pos_hbm, negw_hbm,
             tidx_v, nidx_v, xin_v, prow_v, nrow_v, pos_v,
             sem, sem_i, sem_x, sem_st):
        wid = lax.axis_index("s") * 2 + lax.axis_index("c")
        pbase = wid * bp
        nbase = wid * rb
        cx = pltpu.async_copy(x_hbm.at[pl.ds(pbase, bp)], xin_v, sem_x)
        ci1 = pltpu.async_copy(tgt_hbm.at[pl.ds(pbase, bp)], tidx_v, sem_i)
        ci2 = pltpu.async_copy(ids_hbm.at[pl.ds(nbase, rb)], nidx_v, sem_i)
        ci1.wait()
        ci2.wait()
        c1 = pltpu.async_copy(table_hbm.at[tidx_v], prow_v, sem)
        c2 = pltpu.async_copy(table_hbm.at[nidx_v], nrow_v, sem)
        c1.wait()
        c2.wait()
        cst = pltpu.async_copy(nrow_v, negw_hbm.at[pl.ds(nbase, rb)], sem_st)
        cx.wait()

        lanes = lax.iota(jnp.int32, 16)
        gdn = lax.GatherDimensionNumbers(
            offset_dims=(), collapsed_slice_dims=(0,), start_index_map=(0,))
        perms = [(lanes ^ sh)[:, None] for sh in (8, 4, 2, 1)]

        def group_dot(g, _):
            vec = jnp.zeros((16,), jnp.float32)
            for j in range(16):
                r = g * 16 + j
                acc = prow_v[r, pl.ds(0, 16)] * xin_v[r, pl.ds(0, 16)]
                for c in range(1, nd):
                    acc = acc + (prow_v[r, pl.ds(c * 16, 16)]
                                 * xin_v[r, pl.ds(c * 16, 16)])
                for p in perms:
                    acc = acc + lax.gather(
                        acc, p, dimension_numbers=gdn, slice_sizes=(1,),
                        mode=lax.GatherScatterMode.PROMISE_IN_BOUNDS)
                vec = jnp.where(lanes == j, acc, vec)
            pos_v[pl.ds(g * 16, 16)] = vec
            return _

        lax.fori_loop(0, bp // 16, group_dot, 0)
        pltpu.sync_copy(pos_v, pos_hbm.at[pl.ds(pbase, bp)])
        cst.wait()

    return sc_b


def _tc_a_body(mids_ref, tgt_ref, x_ref, nw_ref, out_ref):
    scores = lax.dot_general(nw_ref[...], x_ref[...], (((1,), (1,)), ((), ())),
                             preferred_element_type=jnp.float32)
    mask = mids_ref[...] == tgt_ref[...]
    out_ref[...] = jnp.where(mask, MIN_FLOAT, scores)


def _tc_b_body(mids_ref, tgt_ref, pos_ref, x_ref, nw_ref, prev_ref, out_ref,
               *, bn, nb, off):
    j = pl.program_id(0)
    scores = lax.dot_general(nw_ref[...], x_ref[...], (((1,), (1,)), ((), ())),
                             preferred_element_type=jnp.float32)
    mask = mids_ref[...] == tgt_ref[...]
    scores = jnp.where(mask, MIN_FLOAT, scores)
    row0 = jnp.where(j == nb - 1, 0, (j + off) * bn)
    grow = row0 + lax.broadcasted_iota(jnp.int32, scores.shape, 0)
    scores = jnp.where(grow == 0, pos_ref[...], scores)
    out_ref[...] = scores


def kernel(inputs, targets, neg_ids, table):
    B, D = inputs.shape
    V = table.shape[0]
    N = neg_ids.shape[0]
    BN = 256
    tgt = targets.astype(jnp.int32)
    nid = neg_ids.astype(jnp.int32)

    NPAD = ((N + 1 + 255) // 256) * 256
    tail = NPAD - 1 - N
    nid_pad = jnp.concatenate(
        [jnp.zeros((1,), jnp.int32), nid, jnp.zeros((tail,), jnp.int32)])
    mids_col = jnp.concatenate(
        [jnp.full((1,), -1, jnp.int32), nid, jnp.full((tail,), -1, jnp.int32)]
    ).reshape(NPAD, 1)

    RA = 12 * BN
    ids_a = lax.slice(nid_pad, (BN,), (BN + RA,))
    ids_b = jnp.concatenate(
        [lax.slice(nid_pad, (BN + RA,), (NPAD,)), lax.slice(nid_pad, (0,), (BN,))])
    RB = NPAD - RA
    OFF = RA // BN + 1

    res_a = _make_sc_gather_a(V, D, RA)(ids_a, table)
    negw_a = res_a[0] if isinstance(res_a, (list, tuple)) else res_a
    pos, negw_b = _make_sc_gather_b(V, D, B, RB)(tgt, ids_b, inputs, table)

    tgt_row = tgt.reshape(1, B)
    pos_row = pos.reshape(1, B)
    n_out = N + 1

    out_a = pl.pallas_call(
        _tc_a_body,
        grid=(RA // BN,),
        in_specs=[
            pl.BlockSpec((BN, 1), lambda j: (j + 1, 0)),
            pl.BlockSpec(memory_space=pltpu.VMEM),
            pl.BlockSpec(memory_space=pltpu.VMEM),
            pl.BlockSpec((BN, D), lambda j: (j, 0)),
        ],
        out_specs=pl.BlockSpec((BN, B), lambda j: (j + 1, 0)),
        out_shape=jax.ShapeDtypeStruct((n_out, B), jnp.float32),
        compiler_params=pltpu.CompilerParams(
            dimension_semantics=("arbitrary",)),
    )(mids_col, tgt_row, inputs, negw_a)

    nb = RB // BN
    out_t = pl.pallas_call(
        functools.partial(_tc_b_body, bn=BN, nb=nb, off=OFF),
        grid=(nb,),
        in_specs=[
            pl.BlockSpec((BN, 1),
                         lambda j: (jnp.where(j == nb - 1, 0, j + OFF), 0)),
            pl.BlockSpec(memory_space=pltpu.VMEM),
            pl.BlockSpec(memory_space=pltpu.VMEM),
            pl.BlockSpec(memory_space=pltpu.VMEM),
            pl.BlockSpec((BN, D), lambda j: (j, 0)),
            pl.BlockSpec(memory_space=pl.ANY),
        ],
        out_specs=pl.BlockSpec((BN, B),
                               lambda j: (jnp.where(j == nb - 1, 0, j + OFF), 0)),
        out_shape=jax.ShapeDtypeStruct((n_out, B), jnp.float32),
        input_output_aliases={5: 0},
        compiler_params=pltpu.CompilerParams(
            dimension_semantics=("arbitrary",)),
    )(mids_col, tgt_row, pos_row, inputs, negw_b, out_a)

    return out_t.T


# speedup vs baseline: 1.0802x; 1.0619x over previous
import functools

import jax
import jax.numpy as jnp
from jax import lax
from jax.experimental import pallas as pl
from jax.experimental.pallas import tpu as pltpu
from jax.experimental.pallas import tpu_sc as plsc

MIN_FLOAT = -3.4028234663852886e+36

_NW = 32


def _make_sc_gather(V, D, B, NPAD):
    bp = B // _NW
    np_ = NPAD // _NW
    np_a = min(np_, 128)
    np_b = np_ - np_a
    nd = D // 16

    mesh = plsc.VectorSubcoreMesh(core_axis_name="c", subcore_axis_name="s")

    @functools.partial(
        pl.kernel,
        mesh=mesh,
        out_type=[
            jax.ShapeDtypeStruct((B,), jnp.float32),
            jax.ShapeDtypeStruct((NPAD, D), jnp.float32),
        ],
        scratch_types=[
            pltpu.VMEM((bp,), jnp.int32),
            pltpu.VMEM((np_a,), jnp.int32),
            pltpu.VMEM((max(np_b, 8),), jnp.int32),
            pltpu.VMEM((bp, D), jnp.float32),
            pltpu.VMEM((bp, D), jnp.float32),
            pltpu.VMEM((np_, D), jnp.float32),
            pltpu.VMEM((bp,), jnp.float32),
            pltpu.SemaphoreType.DMA,
            pltpu.SemaphoreType.DMA,
            pltpu.SemaphoreType.DMA,
        ],
    )
    def sc_gather(tgt_hbm, nid_hbm, x_hbm, table_hbm, pos_hbm, negw_hbm,
                  tidx_v, nidx_a, nidx_b, xin_v, prow_v, nrow_v, pos_v,
                  sem, sem_x, sem_st):
        wid = lax.axis_index("s") * 2 + lax.axis_index("c")
        pbase = wid * bp
        nbase = wid * np_
        cx = pltpu.async_copy(x_hbm.at[pl.ds(pbase, bp)], xin_v, sem_x)
        pltpu.sync_copy(tgt_hbm.at[pl.ds(pbase, bp)], tidx_v)
        pltpu.sync_copy(nid_hbm.at[pl.ds(nbase, np_a)], nidx_a)
        if np_b:
            pltpu.sync_copy(nid_hbm.at[pl.ds(nbase + np_a, np_b)],
                            nidx_b.at[pl.ds(0, np_b)])
        c1 = pltpu.async_copy(table_hbm.at[tidx_v], prow_v, sem)
        c2 = pltpu.async_copy(table_hbm.at[nidx_a], nrow_v.at[pl.ds(0, np_a)],
                              sem)
        if np_b:
            c3 = pltpu.async_copy(table_hbm.at[nidx_b.at[pl.ds(0, np_b)]],
                                  nrow_v.at[pl.ds(np_a, np_b)], sem)
        c1.wait()
        c2.wait()
        if np_b:
            c3.wait()
        cst = pltpu.async_copy(nrow_v, negw_hbm.at[pl.ds(nbase, np_)], sem_st)
        cx.wait()

        lanes = lax.iota(jnp.int32, 16)
        gdn = lax.GatherDimensionNumbers(
            offset_dims=(), collapsed_slice_dims=(0,), start_index_map=(0,))
        perms = [(lanes ^ sh)[:, None] for sh in (8, 4, 2, 1)]

        def group_dot(g, _):
            vec = jnp.zeros((16,), jnp.float32)
            for j in range(16):
                r = g * 16 + j
                acc = prow_v[r, pl.ds(0, 16)] * xin_v[r, pl.ds(0, 16)]
                for c in range(1, nd):
                    acc = acc + (prow_v[r, pl.ds(c * 16, 16)]
                                 * xin_v[r, pl.ds(c * 16, 16)])
                for p in perms:
                    acc = acc + lax.gather(
                        acc, p, dimension_numbers=gdn, slice_sizes=(1,),
                        mode=lax.GatherScatterMode.PROMISE_IN_BOUNDS)
                vec = jnp.where(lanes == j, acc, vec)
            pos_v[pl.ds(g * 16, 16)] = vec
            return _

        lax.fori_loop(0, bp // 16, group_dot, 0)
        pltpu.sync_copy(pos_v, pos_hbm.at[pl.ds(pbase, bp)])
        cst.wait()

    return sc_gather


def _tc_score_body(mids_ref, tgt_ref, pos_ref, x_ref, nw_ref, out_ref, *, bn):
    i = pl.program_id(0)
    scores = lax.dot_general(nw_ref[...], x_ref[...], (((1,), (1,)), ((), ())),
                             preferred_element_type=jnp.float32)
    mask = mids_ref[...] == tgt_ref[...]
    scores = jnp.where(mask, MIN_FLOAT, scores)
    grow = i * bn + lax.broadcasted_iota(jnp.int32, scores.shape, 0)
    scores = jnp.where(grow == 0, pos_ref[...], scores)
    out_ref[...] = scores


def _tc_score(inputs, pos_row, negw, tgt_row, mids_col, n_out, bn=512):
    B, D = inputs.shape
    NPAD = negw.shape[0]
    grid = (pl.cdiv(NPAD, bn),)
    return pl.pallas_call(
        functools.partial(_tc_score_body, bn=bn),
        grid=grid,
        in_specs=[
            pl.BlockSpec((bn, 1), lambda i: (i, 0)),
            pl.BlockSpec(memory_space=pltpu.VMEM),
            pl.BlockSpec(memory_space=pltpu.VMEM),
            pl.BlockSpec(memory_space=pltpu.VMEM),
            pl.BlockSpec((bn, D), lambda i: (i, 0)),
        ],
        out_specs=pl.BlockSpec((bn, B), lambda i: (i, 0)),
        out_shape=jax.ShapeDtypeStruct((n_out, B), jnp.float32),
        compiler_params=pltpu.CompilerParams(
            dimension_semantics=("arbitrary",)),
    )(mids_col, tgt_row, pos_row, inputs, negw)


def kernel(inputs, targets, neg_ids, table):
    B, D = inputs.shape
    V = table.shape[0]
    N = neg_ids.shape[0]
    tgt = targets.astype(jnp.int32)
    nid = neg_ids.astype(jnp.int32)

    NPAD = ((N + 1 + 255) // 256) * 256
    tail = NPAD - 1 - N
    nid_pad = jnp.concatenate(
        [jnp.zeros((1,), jnp.int32), nid, jnp.zeros((tail,), jnp.int32)])
    mids_col = jnp.concatenate(
        [jnp.full((1,), -1, jnp.int32), nid, jnp.full((tail,), -1, jnp.int32)]
    ).reshape(NPAD, 1)

    sc_gather = _make_sc_gather(V, D, B, NPAD)
    pos, negw = sc_gather(tgt, nid_pad, inputs, table)

    out_t = _tc_score(inputs, pos.reshape(1, B), negw, tgt.reshape(1, B),
                      mids_col, N + 1)
    return out_t.T


# speedup vs baseline: 1.1316x; 1.0476x over previous
import functools

import jax
import jax.numpy as jnp
from jax import lax
from jax.experimental import pallas as pl
from jax.experimental.pallas import tpu as pltpu
from jax.experimental.pallas import tpu_sc as plsc

MIN_FLOAT = -3.4028234663852886e+36

_NW = 32


def _make_sc_gather(V, D, B, NPAD):
    bp = B // _NW
    np_ = NPAD // _NW
    np_a = min(np_, 128)
    np_b = np_ - np_a
    nd = D // 16

    mesh = plsc.VectorSubcoreMesh(core_axis_name="c", subcore_axis_name="s")

    @functools.partial(
        pl.kernel,
        mesh=mesh,
        out_type=[
            jax.ShapeDtypeStruct((B,), jnp.float32),
            jax.ShapeDtypeStruct((NPAD, D), jnp.float32),
        ],
        scratch_types=[
            pltpu.VMEM((bp,), jnp.int32),
            pltpu.VMEM((np_a,), jnp.int32),
            pltpu.VMEM((max(np_b, 8),), jnp.int32),
            pltpu.VMEM((bp, D), jnp.float32),
            pltpu.VMEM((bp, D), jnp.float32),
            pltpu.VMEM((np_, D), jnp.float32),
            pltpu.VMEM((bp,), jnp.float32),
            pltpu.SemaphoreType.DMA,
            pltpu.SemaphoreType.DMA,
            pltpu.SemaphoreType.DMA,
            pltpu.SemaphoreType.DMA,
            pltpu.SemaphoreType.DMA,
        ],
    )
    def sc_gather(tgt_hbm, nid_hbm, x_hbm, table_hbm, pos_hbm, negw_hbm,
                  tidx_v, nidx_a, nidx_b, xin_v, prow_v, nrow_v, pos_v,
                  sem, sem_x, sem_st, sem_i, sem_p):
        wid = lax.axis_index("s") * 2 + lax.axis_index("c")
        pbase = wid * bp
        nbase = wid * np_
        cx = pltpu.async_copy(x_hbm.at[pl.ds(pbase, bp)], xin_v, sem_x)
        ci1 = pltpu.async_copy(tgt_hbm.at[pl.ds(pbase, bp)], tidx_v, sem_i)
        ci2 = pltpu.async_copy(nid_hbm.at[pl.ds(nbase, np_a)], nidx_a, sem_i)
        if np_b:
            ci3 = pltpu.async_copy(nid_hbm.at[pl.ds(nbase + np_a, np_b)],
                                   nidx_b.at[pl.ds(0, np_b)], sem_i)
        ci1.wait()
        ci2.wait()
        if np_b:
            ci3.wait()
        c1 = pltpu.async_copy(table_hbm.at[tidx_v], prow_v, sem_p)
        c2 = pltpu.async_copy(table_hbm.at[nidx_a], nrow_v.at[pl.ds(0, np_a)],
                              sem)
        if np_b:
            c3 = pltpu.async_copy(table_hbm.at[nidx_b.at[pl.ds(0, np_b)]],
                                  nrow_v.at[pl.ds(np_a, np_b)], sem)
        c1.wait()
        cx.wait()

        lanes = lax.iota(jnp.int32, 16)
        gdn = lax.GatherDimensionNumbers(
            offset_dims=(), collapsed_slice_dims=(0,), start_index_map=(0,))
        perms = [(lanes ^ sh)[:, None] for sh in (8, 4, 2, 1)]

        def group_dot(g, _):
            vec = jnp.zeros((16,), jnp.float32)
            for j in range(16):
                r = g * 16 + j
                acc = prow_v[r, pl.ds(0, 16)] * xin_v[r, pl.ds(0, 16)]
                for c in range(1, nd):
                    acc = acc + (prow_v[r, pl.ds(c * 16, 16)]
                                 * xin_v[r, pl.ds(c * 16, 16)])
                for p in perms:
                    acc = acc + lax.gather(
                        acc, p, dimension_numbers=gdn, slice_sizes=(1,),
                        mode=lax.GatherScatterMode.PROMISE_IN_BOUNDS)
                vec = jnp.where(lanes == j, acc, vec)
            pos_v[pl.ds(g * 16, 16)] = vec
            return _

        lax.fori_loop(0, bp // 16, group_dot, 0)
        c2.wait()
        if np_b:
            c3.wait()
        cst = pltpu.async_copy(nrow_v, negw_hbm.at[pl.ds(nbase, np_)], sem_st)
        pltpu.sync_copy(pos_v, pos_hbm.at[pl.ds(pbase, bp)])
        cst.wait()

    return sc_gather


def _tc_score_body(mids_ref, tgt_ref, pos_ref, x_ref, nw_ref, out_ref, *, bn):
    i = pl.program_id(0)
    scores = lax.dot_general(nw_ref[...], x_ref[...], (((1,), (1,)), ((), ())),
                             preferred_element_type=jnp.float32)
    mask = mids_ref[...] == tgt_ref[...]
    scores = jnp.where(mask, MIN_FLOAT, scores)
    grow = i * bn + lax.broadcasted_iota(jnp.int32, scores.shape, 0)
    scores = jnp.where(grow == 0, pos_ref[...], scores)
    out_ref[...] = scores


def _tc_score(inputs, pos_row, negw, tgt_row, mids_col, n_out, bn=512):
    B, D = inputs.shape
    NPAD = negw.shape[0]
    grid = (pl.cdiv(NPAD, bn),)
    return pl.pallas_call(
        functools.partial(_tc_score_body, bn=bn),
        grid=grid,
        in_specs=[
            pl.BlockSpec((bn, 1), lambda i: (i, 0)),
            pl.BlockSpec(memory_space=pltpu.VMEM),
            pl.BlockSpec(memory_space=pltpu.VMEM),
            pl.BlockSpec(memory_space=pltpu.VMEM),
            pl.BlockSpec((bn, D), lambda i: (i, 0)),
        ],
        out_specs=pl.BlockSpec((bn, B), lambda i: (i, 0)),
        out_shape=jax.ShapeDtypeStruct((n_out, B), jnp.float32),
        compiler_params=pltpu.CompilerParams(
            dimension_semantics=("arbitrary",)),
    )(mids_col, tgt_row, pos_row, inputs, negw)


def kernel(inputs, targets, neg_ids, table):
    B, D = inputs.shape
    V = table.shape[0]
    N = neg_ids.shape[0]
    tgt = targets.astype(jnp.int32)
    nid = neg_ids.astype(jnp.int32)

    NPAD = ((N + 1 + 255) // 256) * 256
    tail = NPAD - 1 - N
    nid_pad = jnp.concatenate(
        [jnp.zeros((1,), jnp.int32), nid, jnp.zeros((tail,), jnp.int32)])
    mids_col = jnp.concatenate(
        [jnp.full((1,), -1, jnp.int32), nid, jnp.full((tail,), -1, jnp.int32)]
    ).reshape(NPAD, 1)

    sc_gather = _make_sc_gather(V, D, B, NPAD)
    pos, negw = sc_gather(tgt, nid_pad, inputs, table)

    out_t = _tc_score(inputs, pos.reshape(1, B), negw, tgt.reshape(1, B),
                      mids_col, N + 1)
    return out_t.T
